# Initial kernel scaffold; baseline (speedup 1.0000x reference)
import jax
import jax.numpy as jnp
from jax.experimental import pallas as pl


def kernel(hidden_states, gate_w, w1, w3, w2):
    raise NotImplementedError("write your pallas kernel here")


# speedup vs baseline: 1.0481x; 1.0481x over previous
import functools

import jax
import jax.numpy as jnp
from jax import lax
from jax.experimental import pallas as pl
from jax.experimental.pallas import tpu as pltpu

S = 2048
H = 2048
E = 8
K = 2
FF = 2048
BM = 128
NP = S * K + E * BM
NB = NP // BM
NBPAD = 48

RT = 256


def _router_body(x_ref, gate_ref, selw_ref, seli_ref):
    x = x_ref[...]
    g = gate_ref[...]
    logits = lax.dot_general(x, g, (((1,), (1,)), ((), ())),
                             preferred_element_type=jnp.float32)
    lane = lax.broadcasted_iota(jnp.int32, logits.shape, 1)
    neg = jnp.float32(-1e30)
    logits = jnp.where(lane < E, logits, neg)
    m = jnp.max(logits, axis=1, keepdims=True)
    p = jnp.exp(logits - m)
    p = jnp.where(lane < E, p, 0.0)
    p = p / jnp.sum(p, axis=1, keepdims=True)
    m1 = jnp.max(p, axis=1, keepdims=True)
    a1 = jnp.min(jnp.where(p == m1, lane, 128), axis=1, keepdims=True)
    p2 = jnp.where(lane == a1, -1.0, p)
    m2 = jnp.max(p2, axis=1, keepdims=True)
    a2 = jnp.min(jnp.where(p2 == m2, lane, 128), axis=1, keepdims=True)
    tot = m1 + m2
    w0 = m1 / tot
    w1 = m2 / tot
    selw_ref[...] = jnp.where(lane == 0, w0, jnp.where(lane == 1, w1, 0.0))
    seli_ref[...] = jnp.where(lane == 0, a1, jnp.where(lane == 1, a2, 0))


def _router(x, gate_pad):
    return pl.pallas_call(
        _router_body,
        grid=(S // RT,),
        in_specs=[pl.BlockSpec((RT, H), lambda i: (i, 0)),
                  pl.BlockSpec((128, H), lambda i: (0, 0))],
        out_specs=[pl.BlockSpec((RT, 128), lambda i: (i, 0)),
                   pl.BlockSpec((RT, 128), lambda i: (i, 0))],
        out_shape=[jax.ShapeDtypeStruct((S, 128), jnp.float32),
                   jax.ShapeDtypeStruct((S, 128), jnp.int32)],
    )(x, gate_pad)


def _meta_jnp(sel0, sel1):
    s = jnp.concatenate([sel0, sel1])
    onehot = (s[:, None] == jnp.arange(E)[None, :]).astype(jnp.int32)
    rank = jnp.take_along_axis(jnp.cumsum(onehot, axis=0) - 1,
                               s[:, None], axis=1)[:, 0]
    counts = jnp.sum(onehot, axis=0)
    nblk = (counts + BM - 1) // BM
    bstart = jnp.concatenate([jnp.zeros((1,), jnp.int32),
                              jnp.cumsum(nblk)])[:E]
    dest = bstart[s] * BM + rank
    blk_exp = jnp.clip(
        jnp.sum(jnp.arange(NB)[:, None] >= bstart[None, :], axis=1) - 1,
        0, E - 1).astype(jnp.int32)
    return dest[:S], dest[S:], blk_exp


def _ffn_body(be_ref, xs_ref, w1_ref, w3_ref, w2_ref, ys_ref):
    del be_ref
    x = xs_ref[...]
    h1 = lax.dot_general(x, w1_ref[0], (((1,), (1,)), ((), ())),
                         preferred_element_type=jnp.float32)
    h3 = lax.dot_general(x, w3_ref[0], (((1,), (1,)), ((), ())),
                         preferred_element_type=jnp.float32)
    h = (h1 * jax.nn.sigmoid(h1)) * h3
    y = lax.dot_general(h.astype(jnp.bfloat16), w2_ref[0],
                        (((1,), (1,)), ((), ())),
                        preferred_element_type=jnp.float32)
    ys_ref[...] = y


def _ffn(blk_exp, xs, w1b, w3b, w2b):
    return pl.pallas_call(
        _ffn_body,
        grid_spec=pltpu.PrefetchScalarGridSpec(
            num_scalar_prefetch=1,
            grid=(NB,),
            in_specs=[pl.BlockSpec((BM, H), lambda i, be: (i, 0)),
                      pl.BlockSpec((1, FF, H), lambda i, be: (be[i], 0, 0)),
                      pl.BlockSpec((1, FF, H), lambda i, be: (be[i], 0, 0)),
                      pl.BlockSpec((1, H, FF), lambda i, be: (be[i], 0, 0))],
            out_specs=pl.BlockSpec((BM, H), lambda i, be: (i, 0)),
        ),
        out_shape=jax.ShapeDtypeStruct((NP, H), jnp.float32),
    )(blk_exp, xs, w1b, w3b, w2b)


def kernel(hidden_states, gate_w, w1, w3, w2):
    x = hidden_states.reshape(S, H)
    gate_pad = jnp.zeros((128, H), jnp.float32).at[:E].set(gate_w)

    selw, seli = _router(x, gate_pad)
    sel0 = seli[:, 0]
    sel1 = seli[:, 1]
    tw0 = selw[:, 0]
    tw1 = selw[:, 1]

    d0, d1, blk_exp = _meta_jnp(sel0, sel1)

    x_bf = x.astype(jnp.bfloat16)
    xs = jnp.zeros((NP, H), jnp.bfloat16)
    xs = xs.at[d0].set(x_bf)
    xs = xs.at[d1].set(x_bf)

    ys = _ffn(blk_exp, xs, w1.astype(jnp.bfloat16), w3.astype(jnp.bfloat16),
              w2.astype(jnp.bfloat16))

    out = tw0[:, None] * ys[d0] + tw1[:, None] * ys[d1]
    return out.reshape(hidden_states.shape)
